# baseline (speedup 1.0000x reference)
import functools

import jax
import jax.numpy as jnp
from jax import lax
from jax.experimental import pallas as pl
from jax.experimental.pallas import tpu as pltpu
from jax.experimental.pallas import tpu_sc as plsc

VOCAB = 1000000
EMBED = 64
B = 4096
L = 200

NC = 2
NS = 16
NW = NC * NS

TOTAL = B * L
PER_W = TOTAL // NW
CHUNK = 128
CPW = PER_W // CHUNK


def _make_kernel():
  mesh = plsc.VectorSubcoreMesh(
      core_axis_name="c", subcore_axis_name="s", num_cores=NC, num_subcores=NS
  )

  @functools.partial(
      pl.kernel,
      mesh=mesh,
      compiler_params=pltpu.CompilerParams(use_tc_tiling_on_sc=False),
      out_type=jax.ShapeDtypeStruct((TOTAL, EMBED), jnp.float32),
      scratch_types=[
          pltpu.VMEM((CPW, CHUNK), jnp.int32),
          pltpu.VMEM((CHUNK, EMBED), jnp.float32),
          pltpu.SemaphoreType.DMA,
      ],
  )
  def k(idx_hbm, table_hbm, out_hbm, idx_v, rows_v, sem):
    wid = lax.axis_index("s") * NC + lax.axis_index("c")
    pltpu.sync_copy(idx_hbm.at[pl.ds(wid * CPW, CPW)], idx_v)
    out_base = wid * PER_W

    def step(j, carry):
      pltpu.async_copy(table_hbm.at[idx_v.at[j]], rows_v, sem).wait()
      pltpu.sync_copy(rows_v, out_hbm.at[pl.ds(out_base + j * CHUNK, CHUNK)])
      return carry

    lax.fori_loop(0, CPW, step, 0)

  return k


_kernel_call = _make_kernel()


def kernel(token_ids, embeddings):
  idx = token_ids.reshape(NW * CPW, CHUNK).astype(jnp.int32)
  out = _kernel_call(idx, embeddings)
  return out.reshape(B, L, EMBED)


# speedup vs baseline: 1.1113x; 1.1113x over previous
import functools

import jax
import jax.numpy as jnp
from jax import lax
from jax.experimental import pallas as pl
from jax.experimental.pallas import tpu as pltpu
from jax.experimental.pallas import tpu_sc as plsc

VOCAB = 1000000
EMBED = 64
B = 4096
L = 200

NC = 2
NS = 16
NW = NC * NS

TOTAL = B * L
PER_W = TOTAL // NW
CHUNK = 128
CPW = PER_W // CHUNK
NBUF = 8
NGROUPS = CPW // NBUF


def _make_kernel():
  mesh = plsc.VectorSubcoreMesh(
      core_axis_name="c", subcore_axis_name="s", num_cores=NC, num_subcores=NS
  )

  @functools.partial(
      pl.kernel,
      mesh=mesh,
      compiler_params=pltpu.CompilerParams(use_tc_tiling_on_sc=False),
      out_type=jax.ShapeDtypeStruct((TOTAL, EMBED), jnp.float32),
      scratch_types=[
          pltpu.VMEM((CPW, CHUNK), jnp.int32),
          pltpu.VMEM((NBUF, CHUNK, EMBED), jnp.float32),
          pltpu.SemaphoreType.DMA((NBUF,)),
          pltpu.SemaphoreType.DMA((NBUF,)),
      ],
  )
  def k(idx_hbm, table_hbm, out_hbm, idx_v, rows_v, gsem, ssem):
    wid = lax.axis_index("s") * NC + lax.axis_index("c")
    pltpu.sync_copy(idx_hbm.at[pl.ds(wid * CPW, CPW)], idx_v)
    out_base = wid * PER_W

    def gdesc(j, b):
      return pltpu.make_async_copy(
          table_hbm.at[idx_v.at[j]], rows_v.at[b], gsem.at[b])

    def sdesc(j, b):
      return pltpu.make_async_copy(
          rows_v.at[b], out_hbm.at[pl.ds(out_base + j * CHUNK, CHUNK)],
          ssem.at[b])

    for b in range(NBUF):
      gdesc(b, b).start()

    def group(g, carry):
      for b in range(NBUF):
        j = g * NBUF + b
        gdesc(j, b).wait()
        sdesc(j, b).start()
      @pl.when(g < NGROUPS - 1)
      def _():
        for b in range(NBUF):
          j = g * NBUF + b
          sdesc(j, b).wait()
          gdesc(j + NBUF, b).start()
      return carry

    lax.fori_loop(0, NGROUPS, group, 0)
    for b in range(NBUF):
      sdesc((NGROUPS - 1) * NBUF + b, b).wait()

  return k


_kernel_call = _make_kernel()


def kernel(token_ids, embeddings):
  idx = token_ids.reshape(NW * CPW, CHUNK).astype(jnp.int32)
  out = _kernel_call(idx, embeddings)
  return out.reshape(B, L, EMBED)
